# Initial kernel scaffold; baseline (speedup 1.0000x reference)
import jax
import jax.numpy as jnp
from jax.experimental import pallas as pl


def kernel(input_ids, token_type_ids, word_embeddings, token_type_embeddings, ln_gamma, ln_beta):
    raise NotImplementedError("write your pallas kernel here")


# speedup vs baseline: 3.2234x; 3.2234x over previous
import functools

import jax
import jax.numpy as jnp
from jax import lax
from jax.experimental import pallas as pl
from jax.experimental.pallas import tpu as pltpu
from jax.experimental.pallas import tpu_sc as plsc

HIDDEN = 128
L = 16
NVEC = HIDDEN // L
EPS = 1e-12

NC = 2
NS = 16
NW = NC * NS

IB = 5
CHUNK = IB * 128


def _rsqrt(v):
    i = plsc.bitcast(v, jnp.int32)
    y = plsc.bitcast(jnp.int32(0x5F3759DF) - (i >> 1), jnp.float32)
    for _ in range(3):
        y = y * (1.5 - 0.5 * v * y * y)
    return y


def _lane_perm(x, perm):
    return lax.gather(
        x, perm[:, None],
        dimension_numbers=lax.GatherDimensionNumbers(
            offset_dims=(), collapsed_slice_dims=(0,), start_index_map=(0,)),
        slice_sizes=(1,),
        mode=lax.GatherScatterMode.PROMISE_IN_BOUNDS)


def _allsum(x, perms):
    for p in perms:
        x = x + _lane_perm(x, p)
    return x


def _sc_body(ids_hbm, tt_hbm, table_hbm, tte_hbm, gam_hbm, bet_hbm, out_hbm,
             idx_v, ttid_v, rows_v, tte_v, gam_v, bet_v, sem, *, rows_per_w):
    wid = lax.axis_index("s") * NC + lax.axis_index("c")
    n_groups = rows_per_w // CHUNK

    pltpu.sync_copy(tte_hbm, tte_v)
    pltpu.sync_copy(gam_hbm, gam_v)
    pltpu.sync_copy(bet_hbm, bet_v)

    a = [tte_v[0, pl.ds(j * L, L)] for j in range(NVEC)]
    d = [tte_v[1, pl.ds(j * L, L)] - a[j] for j in range(NVEC)]
    g = [gam_v[pl.ds(j * L, L)] for j in range(NVEC)]
    b = [bet_v[pl.ds(j * L, L)] for j in range(NVEC)]

    iota = lax.iota(jnp.int32, L)
    perms = [iota ^ k for k in (8, 4, 2, 1)]

    def row_body(r, carry):
        a, d, g, b = carry
        f = ttid_v[pl.ds(r, L)][0].astype(jnp.float32)
        x = [rows_v[r, pl.ds(j * L, L)] + a[j] + f * d[j] for j in range(NVEC)]
        acc = x[0]
        acc2 = x[0] * x[0]
        for j in range(1, NVEC):
            acc = acc + x[j]
            acc2 = acc2 + x[j] * x[j]
        mean = _allsum(acc, perms) * (1.0 / HIDDEN)
        ex2 = _allsum(acc2, perms) * (1.0 / HIDDEN)
        rstd = _rsqrt(ex2 - mean * mean + EPS)
        for j in range(NVEC):
            rows_v[r, pl.ds(j * L, L)] = (x[j] - mean) * rstd * g[j] + b[j]
        return carry

    for grp in range(n_groups):
        row_base = wid * rows_per_w + grp * CHUNK
        pltpu.sync_copy(ids_hbm.at[pl.ds(row_base, CHUNK)], idx_v)
        pltpu.sync_copy(tt_hbm.at[pl.ds(row_base, CHUNK)],
                        ttid_v.at[pl.ds(0, CHUNK)])
        copies = [
            pltpu.async_copy(table_hbm.at[idx_v.at[pl.ds(j * 128, 128)]],
                             rows_v.at[pl.ds(j * 128, 128)], sem)
            for j in range(IB)
        ]
        for c in copies:
            c.wait()
        lax.fori_loop(0, CHUNK, row_body, (a, d, g, b))
        pltpu.sync_copy(rows_v, out_hbm.at[pl.ds(row_base, CHUNK)])


def kernel(input_ids, token_type_ids, word_embeddings, token_type_embeddings,
           ln_gamma, ln_beta):
    batch, seq = input_ids.shape
    n_rows = batch * seq
    rows_per_w = n_rows // NW
    assert rows_per_w % CHUNK == 0

    ids_flat = input_ids.reshape(n_rows).astype(jnp.int32)
    tt_flat = token_type_ids.reshape(n_rows).astype(jnp.int32)

    mesh = plsc.VectorSubcoreMesh(core_axis_name="c", subcore_axis_name="s")
    k = functools.partial(
        pl.kernel,
        out_type=jax.ShapeDtypeStruct((n_rows, HIDDEN), jnp.float32),
        mesh=mesh,
        compiler_params=pltpu.CompilerParams(needs_layout_passes=False),
        scratch_types=[
            pltpu.VMEM((CHUNK,), jnp.int32),
            pltpu.VMEM((CHUNK + L,), jnp.int32),
            pltpu.VMEM((CHUNK, HIDDEN), jnp.float32),
            pltpu.VMEM((2, HIDDEN), jnp.float32),
            pltpu.VMEM((HIDDEN,), jnp.float32),
            pltpu.VMEM((HIDDEN,), jnp.float32),
            pltpu.SemaphoreType.DMA,
        ],
    )(functools.partial(_sc_body, rows_per_w=rows_per_w))

    out = k(ids_flat, tt_flat, word_embeddings, token_type_embeddings,
            ln_gamma, ln_beta)
    return out.reshape(batch, seq, HIDDEN)


# speedup vs baseline: 5.4412x; 1.6881x over previous
import functools

import jax
import jax.numpy as jnp
from jax import lax
from jax.experimental import pallas as pl
from jax.experimental.pallas import tpu as pltpu
from jax.experimental.pallas import tpu_sc as plsc

HIDDEN = 128
L = 16
NVEC = HIDDEN // L
EPS = 1e-12

NC = 2
NS = 16
NW = NC * NS

GPB = 5
CHUNK = GPB * 128


def _rsqrt(v):
    i = plsc.bitcast(v, jnp.int32)
    y = plsc.bitcast(jnp.int32(0x5F3759DF) - (i >> 1), jnp.float32)
    for _ in range(3):
        y = y * (1.5 - 0.5 * v * y * y)
    return y


def _lane_perm(x, perm):
    return lax.gather(
        x, perm[:, None],
        dimension_numbers=lax.GatherDimensionNumbers(
            offset_dims=(), collapsed_slice_dims=(0,), start_index_map=(0,)),
        slice_sizes=(1,),
        mode=lax.GatherScatterMode.PROMISE_IN_BOUNDS)


def _allsum(x, perms):
    for p in perms:
        x = x + _lane_perm(x, p)
    return x


def _sc_body(ids_hbm, tt_hbm, table_hbm, tte_hbm, gam_hbm, bet_hbm, out_hbm,
             idx_v, ttid_v, rows_v, tte_v, gam_v, bet_v, gsem, osem,
             *, rows_per_w):
    wid = lax.axis_index("s") * NC + lax.axis_index("c")
    n_groups = rows_per_w // CHUNK

    pltpu.sync_copy(tte_hbm, tte_v)
    pltpu.sync_copy(gam_hbm, gam_v)
    pltpu.sync_copy(bet_hbm, bet_v)

    a = [tte_v[0, pl.ds(j * L, L)] for j in range(NVEC)]
    d = [tte_v[1, pl.ds(j * L, L)] - a[j] for j in range(NVEC)]
    g = [gam_v[pl.ds(j * L, L)] for j in range(NVEC)]
    b = [bet_v[pl.ds(j * L, L)] for j in range(NVEC)]

    iota = lax.iota(jnp.int32, L)
    perms = [iota ^ k for k in (8, 4, 2, 1)]

    def stage(grp, buf):
        row_base = wid * rows_per_w + grp * CHUNK
        pltpu.sync_copy(ids_hbm.at[pl.ds(row_base, CHUNK)], idx_v.at[buf])
        pltpu.sync_copy(tt_hbm.at[pl.ds(row_base, CHUNK)],
                        ttid_v.at[buf, pl.ds(0, CHUNK)])
        return [
            pltpu.async_copy(
                table_hbm.at[idx_v.at[buf, pl.ds(j * 128, 128)]],
                rows_v.at[buf, pl.ds(j * 128, 128)], gsem.at[buf])
            for j in range(GPB)
        ]

    def compute(buf):
        @plsc.parallel_loop(0, CHUNK, step=1, unroll=4)
        def row_body(r):
            f = ttid_v[buf, pl.ds(r, L)][0].astype(jnp.float32)
            x = [rows_v[buf, r, pl.ds(j * L, L)] + a[j] + f * d[j]
                 for j in range(NVEC)]
            acc = x[0]
            acc2 = x[0] * x[0]
            for j in range(1, NVEC):
                acc = acc + x[j]
                acc2 = acc2 + x[j] * x[j]
            mean = _allsum(acc, perms) * (1.0 / HIDDEN)
            ex2 = _allsum(acc2, perms) * (1.0 / HIDDEN)
            rstd = _rsqrt(ex2 - mean * mean + EPS)
            for j in range(NVEC):
                rows_v[buf, r, pl.ds(j * L, L)] = \
                    (x[j] - mean) * rstd * g[j] + b[j]

    for grp in range(n_groups):
        for c in stage(grp, 0):
            c.wait()
        compute(0)
        row_base = wid * rows_per_w + grp * CHUNK
        pltpu.async_copy(
            rows_v.at[0], out_hbm.at[pl.ds(row_base, CHUNK)], osem.at[0]
        ).wait()


def kernel(input_ids, token_type_ids, word_embeddings, token_type_embeddings,
           ln_gamma, ln_beta):
    batch, seq = input_ids.shape
    n_rows = batch * seq
    rows_per_w = n_rows // NW
    assert rows_per_w % CHUNK == 0

    ids_flat = input_ids.reshape(n_rows).astype(jnp.int32)
    tt_flat = token_type_ids.reshape(n_rows).astype(jnp.int32)

    mesh = plsc.VectorSubcoreMesh(core_axis_name="c", subcore_axis_name="s")
    k = functools.partial(
        pl.kernel,
        out_type=jax.ShapeDtypeStruct((n_rows, HIDDEN), jnp.float32),
        mesh=mesh,
        compiler_params=pltpu.CompilerParams(needs_layout_passes=False),
        scratch_types=[
            pltpu.VMEM((1, CHUNK), jnp.int32),
            pltpu.VMEM((1, CHUNK + L), jnp.int32),
            pltpu.VMEM((1, CHUNK, HIDDEN), jnp.float32),
            pltpu.VMEM((2, HIDDEN), jnp.float32),
            pltpu.VMEM((HIDDEN,), jnp.float32),
            pltpu.VMEM((HIDDEN,), jnp.float32),
            pltpu.SemaphoreType.DMA((1,)),
            pltpu.SemaphoreType.DMA((1,)),
        ],
    )(functools.partial(_sc_body, rows_per_w=rows_per_w))

    out = k(ids_flat, tt_flat, word_embeddings, token_type_embeddings,
            ln_gamma, ln_beta)
    return out.reshape(batch, seq, HIDDEN)


# speedup vs baseline: 6.2782x; 1.1538x over previous
import functools

import jax
import jax.numpy as jnp
from jax import lax
from jax.experimental import pallas as pl
from jax.experimental.pallas import tpu as pltpu
from jax.experimental.pallas import tpu_sc as plsc

HIDDEN = 128
L = 16
NVEC = HIDDEN // L
EPS = 1e-12

NC = 2
NS = 16
NW = NC * NS

CHUNK = 128


def _rsqrt(v):
    i = plsc.bitcast(v, jnp.int32)
    y = plsc.bitcast(jnp.int32(0x5F3759DF) - (i >> 1), jnp.float32)
    for _ in range(3):
        y = y * (1.5 - 0.5 * v * y * y)
    return y


def _lane_perm(x, perm):
    return lax.gather(
        x, perm[:, None],
        dimension_numbers=lax.GatherDimensionNumbers(
            offset_dims=(), collapsed_slice_dims=(0,), start_index_map=(0,)),
        slice_sizes=(1,),
        mode=lax.GatherScatterMode.PROMISE_IN_BOUNDS)


def _allsum(x, perms):
    for p in perms:
        x = x + _lane_perm(x, p)
    return x


def _sc_body(ids_hbm, tt_hbm, table_hbm, tte_hbm, gam_hbm, bet_hbm, out_hbm,
             idx_v, ttid_v, rows_v, outb_v, tte_v, gam_v, bet_v, gsem, osem,
             *, rows_per_w):
    wid = lax.axis_index("s") * NC + lax.axis_index("c")
    n_groups = rows_per_w // CHUNK
    n_pairs = n_groups // 2
    row0 = wid * rows_per_w

    pltpu.sync_copy(tte_hbm, tte_v)
    pltpu.sync_copy(gam_hbm, gam_v)
    pltpu.sync_copy(bet_hbm, bet_v)

    a = [tte_v[0, pl.ds(j * L, L)] for j in range(NVEC)]
    d = [tte_v[1, pl.ds(j * L, L)] - a[j] for j in range(NVEC)]
    g = [gam_v[pl.ds(j * L, L)] for j in range(NVEC)]
    b = [bet_v[pl.ds(j * L, L)] for j in range(NVEC)]

    iota = lax.iota(jnp.int32, L)
    perms = [iota ^ k for k in (8, 4, 2, 1)]

    def stage(grp, buf):
        row_base = row0 + grp * CHUNK
        pltpu.sync_copy(ids_hbm.at[pl.ds(row_base, CHUNK)], idx_v.at[buf])
        pltpu.sync_copy(tt_hbm.at[pl.ds(row_base, CHUNK)],
                        ttid_v.at[buf, pl.ds(0, CHUNK)])
        pltpu.async_copy(table_hbm.at[idx_v.at[buf]], rows_v.at[buf],
                         gsem.at[buf])

    def wait_gather(buf):
        pltpu.make_async_copy(table_hbm.at[idx_v.at[buf]], rows_v.at[buf],
                              gsem.at[buf]).wait()

    def start_out(grp, buf):
        row_base = row0 + grp * CHUNK
        pltpu.async_copy(outb_v.at[buf], out_hbm.at[pl.ds(row_base, CHUNK)],
                         osem.at[buf])

    def wait_out(buf):
        pltpu.make_async_copy(outb_v.at[buf], out_hbm.at[pl.ds(row0, CHUNK)],
                              osem.at[buf]).wait()

    def compute(buf):
        @plsc.parallel_loop(0, CHUNK, step=1, unroll=4)
        def row_body(r):
            f = ttid_v[buf, pl.ds(r, L)][0].astype(jnp.float32)
            x = [rows_v[buf, r, pl.ds(j * L, L)] + a[j] + f * d[j]
                 for j in range(NVEC)]
            acc = x[0]
            acc2 = x[0] * x[0]
            for j in range(1, NVEC):
                acc = acc + x[j]
                acc2 = acc2 + x[j] * x[j]
            mean = _allsum(acc, perms) * (1.0 / HIDDEN)
            ex2 = _allsum(acc2, perms) * (1.0 / HIDDEN)
            rstd = _rsqrt(ex2 - mean * mean + EPS)
            for j in range(NVEC):
                outb_v[buf, r, pl.ds(j * L, L)] = \
                    (x[j] - mean) * rstd * g[j] + b[j]

    stage(0, 0)
    stage(1, 1)

    def pair_body(p, _):
        grp0 = 2 * p

        def half(buf):
            grp = grp0 + buf
            wait_gather(buf)

            @pl.when(p > 0)
            def _():
                wait_out(buf)

            compute(buf)
            start_out(grp, buf)

            @pl.when(p + 1 < n_pairs)
            def _():
                stage(grp + 2, buf)

        half(0)
        half(1)
        return _

    lax.fori_loop(0, n_pairs, pair_body, 0)
    wait_out(0)
    wait_out(1)


def kernel(input_ids, token_type_ids, word_embeddings, token_type_embeddings,
           ln_gamma, ln_beta):
    batch, seq = input_ids.shape
    n_rows = batch * seq
    rows_per_w = n_rows // NW
    assert rows_per_w % (2 * CHUNK) == 0

    ids_flat = input_ids.reshape(n_rows).astype(jnp.int32)
    tt_flat = token_type_ids.reshape(n_rows).astype(jnp.int32)

    mesh = plsc.VectorSubcoreMesh(core_axis_name="c", subcore_axis_name="s")
    k = functools.partial(
        pl.kernel,
        out_type=jax.ShapeDtypeStruct((n_rows, HIDDEN), jnp.float32),
        mesh=mesh,
        compiler_params=pltpu.CompilerParams(needs_layout_passes=False),
        scratch_types=[
            pltpu.VMEM((2, CHUNK), jnp.int32),
            pltpu.VMEM((2, CHUNK + L), jnp.int32),
            pltpu.VMEM((2, CHUNK, HIDDEN), jnp.float32),
            pltpu.VMEM((2, CHUNK, HIDDEN), jnp.float32),
            pltpu.VMEM((2, HIDDEN), jnp.float32),
            pltpu.VMEM((HIDDEN,), jnp.float32),
            pltpu.VMEM((HIDDEN,), jnp.float32),
            pltpu.SemaphoreType.DMA((2,)),
            pltpu.SemaphoreType.DMA((2,)),
        ],
    )(functools.partial(_sc_body, rows_per_w=rows_per_w))

    out = k(ids_flat, tt_flat, word_embeddings, token_type_embeddings,
            ln_gamma, ln_beta)
    return out.reshape(batch, seq, HIDDEN)
